# baseline (speedup 1.0000x reference)
import jax
import jax.numpy as jnp
from jax import lax
from jax.experimental import pallas as pl
from jax.experimental.pallas import tpu as pltpu


def _make_edge_kernel(BB, E, N):
    bf16 = jnp.bfloat16

    def edge_kernel(ei_ref, ef_ref, we1_ref, wae1_ref, ba1_ref, we2_ref,
                    wae2_ref, ba2_ref,
                    s_ref, t_ref, ts_ref, gate1_ref, eatt1_ref, gate2_ref,
                    eatt2_ref):
        iota = lax.broadcasted_iota(jnp.int32, (E, N), 1)
        s = (iota == ei_ref[:, 0:1]).astype(bf16)
        t = (iota == ei_ref[:, 1:2]).astype(bf16)
        s_ref[...] = s
        t_ref[...] = t
        ts_ref[...] = jnp.concatenate([t, s], axis=1)
        ef = ef_ref[...]
        ee1 = jnp.dot(ef, we1_ref[...], preferred_element_type=jnp.float32)
        gate1_ref[...] = jax.nn.sigmoid(ee1)
        e1 = (jnp.dot(ee1, wae1_ref[...], preferred_element_type=jnp.float32)
              + ba1_ref[...])
        eatt1_ref[...] = jnp.broadcast_to(e1, (E, BB))
        ee2 = jnp.dot(ef, we2_ref[...], preferred_element_type=jnp.float32)
        gate2_ref[...] = jax.nn.sigmoid(ee2)
        e2 = (jnp.dot(ee2, wae2_ref[...], preferred_element_type=jnp.float32)
              + ba2_ref[...])
        eatt2_ref[...] = jnp.broadcast_to(e2, (E, BB))

    return edge_kernel


def _make_main(BB, H, O):
    bf16 = jnp.bfloat16
    f32 = jnp.float32

    def main(x_ref, s_ref, t_ref, ts_ref,
             gate1_ref, eatt1_ref, gate2_ref, eatt2_ref,
             v1_ref, kq1_ref, wc1_ref, bc1_ref,
             v2_ref, kq2_ref, wc2_ref, bc2_ref,
             out_ref):
        S = s_ref[...]
        T = t_ref[...]
        TS = ts_ref[...]

        def layer(h_list, vp, kq, gate, eatt, wcp, bcp, CH):
            xv_pairs = [
                jnp.dot(jnp.concatenate([h_list[2 * p], h_list[2 * p + 1]],
                                        axis=1),
                        vp, preferred_element_type=f32)
                for p in range(BB // 2)]
            xvs = [xv_pairs[b // 2][:, (b % 2) * CH:(b % 2 + 1) * CH]
                   for b in range(BB)]
            xvs_b16 = [x.astype(bf16) for x in xvs]
            xv_cat = jnp.concatenate(xvs_b16, axis=1)
            akq = jnp.dot(xv_cat, kq, preferred_element_type=f32)
            kqs = jnp.concatenate([akq[:, :BB], akq[:, BB:]], axis=0)
            logits = jnp.dot(TS, kqs.astype(bf16),
                             preferred_element_type=f32) + eatt
            att = jax.nn.sigmoid(logits)
            xj = jnp.dot(S, xv_cat, preferred_element_type=f32)
            msg = jnp.concatenate(
                [(att[:, b:b + 1] * gate
                  * xj[:, b * CH:(b + 1) * CH]).astype(bf16)
                 for b in range(BB)], axis=1)
            aggr = lax.dot_general(
                T, msg, dimension_numbers=(((0,), (0,)), ((), ())),
                preferred_element_type=f32)
            aggr_b16 = aggr.astype(bf16)
            outs = []
            for p in range(BB // 2):
                b0, b1 = 2 * p, 2 * p + 1
                cat_p = jnp.concatenate(
                    [xvs_b16[b0], aggr_b16[:, b0 * CH:(b0 + 1) * CH],
                     xvs_b16[b1], aggr_b16[:, b1 * CH:(b1 + 1) * CH]],
                    axis=1)
                upd = jnp.dot(cat_p, wcp, preferred_element_type=f32) + bcp
                outs.append(xvs[b0] + jnp.maximum(upd[:, :CH], 0.0))
                outs.append(xvs[b1] + jnp.maximum(upd[:, CH:], 0.0))
            return outs

        h0 = [x_ref[b].astype(bf16) for b in range(BB)]
        h1 = layer(h0, v1_ref[...], kq1_ref[...], gate1_ref[...],
                   eatt1_ref[...], wc1_ref[...], bc1_ref[...], H)
        h1_b16 = [jnp.maximum(h, 0.01 * h).astype(bf16) for h in h1]
        h2 = layer(h1_b16, v2_ref[...], kq2_ref[...], gate2_ref[...],
                   eatt2_ref[...], wc2_ref[...], bc2_ref[...], O)
        for b in range(BB):
            out_ref[b] = h2[b]

    return main


def kernel(X, edge_index, edge_weight, v1, k1, q1, we1, wa1, ba1, wc1, bc1,
           v2, k2, q2, we2, wa2, ba2, wc2, bc2):
    B, N, Cin = X.shape
    E, Ce = edge_weight.shape
    H = v1.shape[1]
    O = v2.shape[1]
    BB = 8 if B % 8 == 0 else (4 if B % 4 == 0 else 2)
    f32 = jnp.float32
    bf16 = jnp.bfloat16
    eye2 = jnp.eye(2, dtype=f32)

    kv1 = k1 @ wa1[:H]
    qv1 = q1 @ wa1[H:2 * H]
    wae1 = wa1[2 * H:]
    kv2 = k2 @ wa2[:O]
    qv2 = q2 @ wa2[O:2 * O]
    wae2 = wa2[2 * O:]

    eye = jnp.eye(BB, dtype=f32)
    kq1 = jnp.concatenate(
        [jnp.kron(eye, kv1), jnp.kron(eye, qv1)], axis=1).astype(bf16)
    kq2 = jnp.concatenate(
        [jnp.kron(eye, kv2), jnp.kron(eye, qv2)], axis=1).astype(bf16)

    S, T, TS, gate1, eatt1, gate2, eatt2 = pl.pallas_call(
        _make_edge_kernel(BB, E, N),
        out_shape=(
            jax.ShapeDtypeStruct((E, N), bf16),
            jax.ShapeDtypeStruct((E, N), bf16),
            jax.ShapeDtypeStruct((E, 2 * N), bf16),
            jax.ShapeDtypeStruct((E, H), f32),
            jax.ShapeDtypeStruct((E, BB), f32),
            jax.ShapeDtypeStruct((E, O), f32),
            jax.ShapeDtypeStruct((E, BB), f32),
        ),
    )(edge_index.T, edge_weight, we1, wae1, ba1, we2, wae2, ba2)

    v1p = jnp.kron(eye2, v1).astype(bf16)
    v2p = jnp.kron(eye2, v2).astype(bf16)
    wc1p = jnp.kron(eye2, wc1).astype(bf16)
    wc2p = jnp.kron(eye2, wc2).astype(bf16)
    bc1p = jnp.concatenate([bc1, bc1], axis=1)
    bc2p = jnp.concatenate([bc2, bc2], axis=1)

    args = (X, S, T, TS, gate1, eatt1, gate2, eatt2,
            v1p, kq1, wc1p, bc1p,
            v2p, kq2, wc2p, bc2p)

    def full(a):
        n = a.ndim
        return pl.BlockSpec(a.shape, lambda i: (0,) * n)

    in_specs = [pl.BlockSpec((BB, N, Cin), lambda i: (i, 0, 0))]
    in_specs += [full(a) for a in args[1:]]

    flops = 2 * B * (2 * E * N * (H + O) + N * Cin * H + N * H * O
                     + 2 * N * H * H + 2 * N * O * O)
    transcendentals = 2 * E * (H + O) + 2 * B * E
    bytes_accessed = (4 * X.size + 4 * B * N * O
                      + 2 * (S.size + T.size + TS.size)
                      + 4 * (gate1.size + gate2.size))

    out = pl.pallas_call(
        _make_main(BB, H, O),
        out_shape=jax.ShapeDtypeStruct((B, N, O), f32),
        grid=(B // BB,),
        in_specs=in_specs,
        out_specs=pl.BlockSpec((BB, N, O), lambda i: (i, 0, 0)),
        compiler_params=pltpu.CompilerParams(
            dimension_semantics=("parallel",)),
        cost_estimate=pl.CostEstimate(
            flops=flops, transcendentals=transcendentals,
            bytes_accessed=bytes_accessed),
    )(*args)
    return out


# speedup vs baseline: 1.0866x; 1.0866x over previous
import jax
import jax.numpy as jnp
from jax import lax
from jax.experimental import pallas as pl
from jax.experimental.pallas import tpu as pltpu


def _make_edge_kernel(BB, E, N):
    bf16 = jnp.bfloat16

    def edge_kernel(ei_ref, ef_ref, we1_ref, wae1_ref, ba1_ref, we2_ref,
                    wae2_ref, ba2_ref,
                    s_ref, t_ref, gate1_ref, eatt1_ref, gate2_ref,
                    eatt2_ref):
        iota = lax.broadcasted_iota(jnp.int32, (E, N), 1)
        s = (iota == ei_ref[:, 0:1]).astype(bf16)
        t = (iota == ei_ref[:, 1:2]).astype(bf16)
        s_ref[...] = s
        t_ref[...] = t
        ef = ef_ref[...]
        ee1 = jnp.dot(ef, we1_ref[...], preferred_element_type=jnp.float32)
        gate1_ref[...] = jax.nn.sigmoid(ee1)
        e1 = (jnp.dot(ee1, wae1_ref[...], preferred_element_type=jnp.float32)
              + ba1_ref[...])
        eatt1_ref[...] = jnp.broadcast_to(e1, (E, BB))
        ee2 = jnp.dot(ef, we2_ref[...], preferred_element_type=jnp.float32)
        gate2_ref[...] = jax.nn.sigmoid(ee2)
        e2 = (jnp.dot(ee2, wae2_ref[...], preferred_element_type=jnp.float32)
              + ba2_ref[...])
        eatt2_ref[...] = jnp.broadcast_to(e2, (E, BB))

    return edge_kernel


def _make_main(BB, H, O):
    bf16 = jnp.bfloat16
    f32 = jnp.float32

    def main(x_ref, s_ref, t_ref,
             gate1_ref, eatt1_ref, gate2_ref, eatt2_ref,
             v1_ref, kq1_ref, wc1_ref, bc1_ref,
             v2_ref, kq2_ref, wc2_ref, bc2_ref,
             out_ref):
        S = s_ref[...]
        T = t_ref[...]

        def layer(h_list, vp, kq, gate, eatt, wcp, bcp, CH):
            xv_pairs = [
                jnp.dot(jnp.concatenate([h_list[2 * p], h_list[2 * p + 1]],
                                        axis=1),
                        vp, preferred_element_type=f32)
                for p in range(BB // 2)]
            xvs = [xv_pairs[b // 2][:, (b % 2) * CH:(b % 2 + 1) * CH]
                   for b in range(BB)]
            xvs_b16 = [x.astype(bf16) for x in xvs]
            xv_cat = jnp.concatenate(xvs_b16, axis=1)
            akq = jnp.dot(xv_cat, kq, preferred_element_type=f32)
            akq_b16 = akq.astype(bf16)
            logits = (jnp.dot(T, akq_b16[:, :BB], preferred_element_type=f32)
                      + jnp.dot(S, akq_b16[:, BB:], preferred_element_type=f32)
                      + eatt)
            att = jax.nn.sigmoid(logits)
            xj = jnp.dot(S, xv_cat, preferred_element_type=f32)
            msg = jnp.concatenate(
                [(att[:, b:b + 1] * gate
                  * xj[:, b * CH:(b + 1) * CH]).astype(bf16)
                 for b in range(BB)], axis=1)
            aggr = lax.dot_general(
                T, msg, dimension_numbers=(((0,), (0,)), ((), ())),
                preferred_element_type=f32)
            aggr_b16 = aggr.astype(bf16)
            outs = []
            for p in range(BB // 2):
                b0, b1 = 2 * p, 2 * p + 1
                cat_p = jnp.concatenate(
                    [xvs_b16[b0], aggr_b16[:, b0 * CH:(b0 + 1) * CH],
                     xvs_b16[b1], aggr_b16[:, b1 * CH:(b1 + 1) * CH]],
                    axis=1)
                upd = jnp.dot(cat_p, wcp, preferred_element_type=f32) + bcp
                outs.append(xvs[b0] + jnp.maximum(upd[:, :CH], 0.0))
                outs.append(xvs[b1] + jnp.maximum(upd[:, CH:], 0.0))
            return outs

        h0 = [x_ref[b].astype(bf16) for b in range(BB)]
        h1 = layer(h0, v1_ref[...], kq1_ref[...], gate1_ref[...],
                   eatt1_ref[...], wc1_ref[...], bc1_ref[...], H)
        h1_b16 = [jnp.maximum(h, 0.01 * h).astype(bf16) for h in h1]
        h2 = layer(h1_b16, v2_ref[...], kq2_ref[...], gate2_ref[...],
                   eatt2_ref[...], wc2_ref[...], bc2_ref[...], O)
        for b in range(BB):
            out_ref[b] = h2[b]

    return main


def kernel(X, edge_index, edge_weight, v1, k1, q1, we1, wa1, ba1, wc1, bc1,
           v2, k2, q2, we2, wa2, ba2, wc2, bc2):
    B, N, Cin = X.shape
    E, Ce = edge_weight.shape
    H = v1.shape[1]
    O = v2.shape[1]
    BB = 16 if B % 16 == 0 else (8 if B % 8 == 0 else (4 if B % 4 == 0 else 2))
    f32 = jnp.float32
    bf16 = jnp.bfloat16
    eye2 = jnp.eye(2, dtype=f32)

    kv1 = k1 @ wa1[:H]
    qv1 = q1 @ wa1[H:2 * H]
    wae1 = wa1[2 * H:]
    kv2 = k2 @ wa2[:O]
    qv2 = q2 @ wa2[O:2 * O]
    wae2 = wa2[2 * O:]

    eye = jnp.eye(BB, dtype=f32)
    kq1 = jnp.concatenate(
        [jnp.kron(eye, kv1), jnp.kron(eye, qv1)], axis=1).astype(bf16)
    kq2 = jnp.concatenate(
        [jnp.kron(eye, kv2), jnp.kron(eye, qv2)], axis=1).astype(bf16)

    S, T, gate1, eatt1, gate2, eatt2 = pl.pallas_call(
        _make_edge_kernel(BB, E, N),
        out_shape=(
            jax.ShapeDtypeStruct((E, N), bf16),
            jax.ShapeDtypeStruct((E, N), bf16),
            jax.ShapeDtypeStruct((E, H), f32),
            jax.ShapeDtypeStruct((E, BB), f32),
            jax.ShapeDtypeStruct((E, O), f32),
            jax.ShapeDtypeStruct((E, BB), f32),
        ),
    )(edge_index.T, edge_weight, we1, wae1, ba1, we2, wae2, ba2)

    v1p = jnp.kron(eye2, v1).astype(bf16)
    v2p = jnp.kron(eye2, v2).astype(bf16)
    wc1p = jnp.kron(eye2, wc1).astype(bf16)
    wc2p = jnp.kron(eye2, wc2).astype(bf16)
    bc1p = jnp.concatenate([bc1, bc1], axis=1)
    bc2p = jnp.concatenate([bc2, bc2], axis=1)

    args = (X, S, T, gate1, eatt1, gate2, eatt2,
            v1p, kq1, wc1p, bc1p,
            v2p, kq2, wc2p, bc2p)

    def full(a):
        n = a.ndim
        return pl.BlockSpec(a.shape, lambda i: (0,) * n)

    in_specs = [pl.BlockSpec((BB, N, Cin), lambda i: (i, 0, 0))]
    in_specs += [full(a) for a in args[1:]]

    flops = 2 * B * (2 * E * N * (H + O) + N * Cin * H + N * H * O
                     + 2 * N * H * H + 2 * N * O * O)
    transcendentals = 2 * E * (H + O) + 2 * B * E
    bytes_accessed = (4 * X.size + 4 * B * N * O + 2 * (S.size + T.size)
                      + 4 * (gate1.size + gate2.size))

    out = pl.pallas_call(
        _make_main(BB, H, O),
        out_shape=jax.ShapeDtypeStruct((B, N, O), f32),
        grid=(B // BB,),
        in_specs=in_specs,
        out_specs=pl.BlockSpec((BB, N, O), lambda i: (i, 0, 0)),
        compiler_params=pltpu.CompilerParams(
            dimension_semantics=("parallel",)),
        cost_estimate=pl.CostEstimate(
            flops=flops, transcendentals=transcendentals,
            bytes_accessed=bytes_accessed),
    )(*args)
    return out
